# baseline (speedup 1.0000x reference)
import jax
import jax.numpy as jnp
from jax import lax
from jax.experimental import pallas as pl
from jax.experimental.pallas import tpu as pltpu
from jax.experimental.pallas import tpu_sc as plsc

VOCAB = 1000000
EMBED_DIM = 32
BATCH = 16384
HIST = 200

_NC = 2
_NS = 16
_NW = _NC * _NS
_BW = BATCH // _NW
_DEPTH = 4


def _embed_body(idx_hbm, tbl_hbm, out3,
                iv0, iv1, iv2, iv3, rv0, rv1, rv2, rv3,
                si0, si1, si2, si3, sg0, sg1, sg2, sg3,
                so0, so1, so2, so3):
    cid = lax.axis_index("c")
    sid = lax.axis_index("s")
    idx_v = [iv0, iv1, iv2, iv3]
    rows = [rv0, rv1, rv2, rv3]
    si = [si0, si1, si2, si3]
    sg = [sg0, sg1, sg2, sg3]
    so = [so0, so1, so2, so3]

    wid = sid * _NC + cid
    b0 = wid * _BW

    def gather(b):
        pltpu.async_copy(tbl_hbm.at[idx_v[b]], rows[b], sg[b])

    def wait_gather(b):
        pltpu.make_async_copy(tbl_hbm.at[idx_v[b]], rows[b], sg[b]).wait()

    def write(i, b):
        pltpu.async_copy(rows[b], out3.at[i, pl.ds(b0, _BW), :], so[b])

    def wait_write(i, b):
        pltpu.make_async_copy(rows[b], out3.at[i, pl.ds(b0, _BW), :],
                              so[b]).wait()

    def load_idx(i, b):
        pltpu.async_copy(idx_hbm.at[pl.ds(i * BATCH + b0, _BW)], idx_v[b],
                         si[b])

    def wait_idx(i, b):
        pltpu.make_async_copy(idx_hbm.at[pl.ds(i * BATCH + b0, _BW)],
                              idx_v[b], si[b]).wait()

    for b in range(_DEPTH):
        load_idx(b, b)
    for b in range(_DEPTH):
        wait_idx(b, b)
        gather(b)
    for b in range(_DEPTH):
        i = b
        wait_gather(b)
        write(i, b)
        load_idx(i + _DEPTH, b)
        wait_idx(i + _DEPTH, b)
        wait_write(i, b)
        gather(b)

    def steady(g, c):
        for b in range(_DEPTH):
            i = g * _DEPTH + b
            wait_gather(b)
            write(i, b)
            load_idx(i + _DEPTH, b)
            wait_idx(i + _DEPTH, b)
            wait_write(i, b)
            gather(b)
        return c

    lax.fori_loop(1, HIST // _DEPTH - 1, steady, 0, unroll=False)

    for b in range(_DEPTH):
        i = HIST - _DEPTH + b
        wait_gather(b)
        write(i, b)
    for b in range(_DEPTH):
        wait_write(HIST - _DEPTH + b, b)


@jax.jit
def _embed(idx, tbl):
    fn = pl.kernel(
        _embed_body,
        mesh=plsc.VectorSubcoreMesh(core_axis_name="c", subcore_axis_name="s"),
        out_type=jax.ShapeDtypeStruct((HIST, BATCH, EMBED_DIM), jnp.float32),
        scratch_types=(
            [pltpu.VMEM((_BW,), jnp.int32) for _ in range(_DEPTH)]
            + [pltpu.VMEM((_BW, EMBED_DIM), jnp.float32)
               for _ in range(_DEPTH)]
            + [pltpu.SemaphoreType.DMA for _ in range(3 * _DEPTH)]
        ),
        compiler_params=pltpu.CompilerParams(use_tc_tiling_on_sc=False,
                                             needs_layout_passes=False),
    )
    return fn(idx, tbl)


def kernel(x, weight):
    idx = x.T.reshape(-1).astype(jnp.int32)
    out3 = _embed(idx, weight)
    return out3.transpose(1, 0, 2)


# speedup vs baseline: 1.2234x; 1.2234x over previous
import jax
import jax.numpy as jnp
from jax import lax
from jax.experimental import pallas as pl
from jax.experimental.pallas import tpu as pltpu
from jax.experimental.pallas import tpu_sc as plsc

VOCAB = 1000000
EMBED_DIM = 32
BATCH = 16384
HIST = 200

_NC = 2
_NS = 16
_NW = _NC * _NS
_BW = BATCH // _NW
_CBW = _BW // 128


def _transpose_chunk(rows, tbuf, iota):
    @plsc.parallel_loop(0, EMBED_DIM, unroll=4)
    def _(d):
        rd = d // 8
        sd = d % 8
        col = jnp.full((16,), d, jnp.int32)
        for cb in range(4):
            for s in range(8):
                rids = iota + (cb * 128 + 16 * s)
                tbuf[rd, cb, sd, pl.ds(16 * s, 16)] = plsc.load_gather(
                    rows, [rids, col])


def _embed_body(idx_hbm, tbl_hbm, out5,
                iv0, iv1, iv2, iv3, rv0, rv1, rv2, rv3, tb0, tb1,
                si0, si1, si2, si3, sg0, sg1, sg2, sg3, so0, so1):
    cid = lax.axis_index("c")
    sid = lax.axis_index("s")
    iota = lax.iota(jnp.int32, 16)
    idx_v = [iv0, iv1, iv2, iv3]
    rows = [rv0, rv1, rv2, rv3]
    tbuf = [tb0, tb1]
    si = [si0, si1, si2, si3]
    sg = [sg0, sg1, sg2, sg3]
    so = [so0, so1]

    wid = sid * _NC + cid
    b0 = wid * _BW
    cb0 = wid * _CBW

    def gather(b):
        pltpu.async_copy(tbl_hbm.at[idx_v[b]], rows[b], sg[b])

    def wait_gather(b):
        pltpu.make_async_copy(tbl_hbm.at[idx_v[b]], rows[b], sg[b]).wait()

    def write(i, b):
        pltpu.async_copy(tbuf[b], out5.at[i, :, pl.ds(cb0, _CBW), :, :],
                         so[b])

    def wait_write(i, b):
        pltpu.make_async_copy(
            tbuf[b], out5.at[i, :, pl.ds(cb0, _CBW), :, :], so[b]).wait()

    def load_idx(i, b):
        pltpu.async_copy(idx_hbm.at[pl.ds(i * BATCH + b0, _BW)], idx_v[b],
                         si[b])

    def wait_idx(i, b):
        pltpu.make_async_copy(idx_hbm.at[pl.ds(i * BATCH + b0, _BW)],
                              idx_v[b], si[b]).wait()

    for b in range(4):
        load_idx(b, b)
    for b in range(4):
        wait_idx(b, b)
        gather(b)
    for b in range(4):
        i = b
        b2 = b % 2
        wait_gather(b)
        load_idx(i + 4, b)
        if i >= 2:
            wait_write(i - 2, b2)
        _transpose_chunk(rows[b], tbuf[b2], iota)
        write(i, b2)
        wait_idx(i + 4, b)
        gather(b)

    def steady(g, c):
        for b in range(4):
            i = g * 4 + b
            b2 = b % 2
            wait_gather(b)
            load_idx(i + 4, b)
            wait_write(i - 2, b2)
            _transpose_chunk(rows[b], tbuf[b2], iota)
            write(i, b2)
            wait_idx(i + 4, b)
            gather(b)
        return c

    lax.fori_loop(1, HIST // 4 - 1, steady, 0, unroll=False)

    for b in range(4):
        i = HIST - 4 + b
        b2 = b % 2
        wait_gather(b)
        wait_write(i - 2, b2)
        _transpose_chunk(rows[b], tbuf[b2], iota)
        write(i, b2)
    for b in range(2):
        wait_write(HIST - 2 + b, b)


@jax.jit
def _embed(idx, tbl):
    fn = pl.kernel(
        _embed_body,
        mesh=plsc.VectorSubcoreMesh(core_axis_name="c", subcore_axis_name="s"),
        out_type=jax.ShapeDtypeStruct((HIST, 4, 128, 8, 128), jnp.float32),
        scratch_types=(
            [pltpu.VMEM((_BW,), jnp.int32) for _ in range(4)]
            + [pltpu.VMEM((_BW, EMBED_DIM), jnp.float32) for _ in range(4)]
            + [pltpu.VMEM((4, _CBW, 8, 128), jnp.float32) for _ in range(2)]
            + [pltpu.SemaphoreType.DMA for _ in range(10)]
        ),
        compiler_params=pltpu.CompilerParams(use_tc_tiling_on_sc=False,
                                             needs_layout_passes=False),
    )
    return fn(idx, tbl)


def kernel(x, weight):
    idx = x.T.reshape(-1).astype(jnp.int32)
    out5 = _embed(idx, weight)
    return out5.transpose(2, 4, 0, 1, 3).reshape(BATCH, HIST, EMBED_DIM)
